# Initial kernel scaffold; baseline (speedup 1.0000x reference)
import jax
import jax.numpy as jnp
from jax.experimental import pallas as pl


def kernel(inputs):
    raise NotImplementedError("write your pallas kernel here")


# speedup vs baseline: 3.1445x; 3.1445x over previous
import functools

import jax
import jax.numpy as jnp
from jax import lax
from jax.experimental import pallas as pl
from jax.experimental.pallas import tpu as pltpu
from jax.experimental.pallas import tpu_sc as plsc

ROWS = 128
N = 32768
LANES = 16
NV = N // LANES
NC = 2
NS = 16
ROWS_PER_WORKER = ROWS // (NC * NS)

def _to_monotone(x):
  m = jnp.int32(-2147483648)
  s = lax.shift_right_arithmetic(x, 31)
  return lax.bitwise_xor(x, lax.bitwise_or(s, m))


def _from_monotone(u):
  m = jnp.int32(-2147483648)
  s = lax.shift_right_arithmetic(u, 31)
  return lax.bitwise_xor(u, lax.bitwise_or(lax.bitwise_not(s), m))


def _zero16():
  return jnp.zeros((LANES,), jnp.int32)


def _sort_body(in_hbm, out_hbm, k0, k1, hist, offs):
  wid = lax.axis_index("s") * NC + lax.axis_index("c")

  def zero_hist(i, c):
    hist[pl.ds(i * LANES, LANES)] = _zero16()
    return c

  def scan_hist(i, carry):
    h = hist[pl.ds(i * LANES, LANES)]
    inc = plsc.cumsum(h)
    offs[pl.ds(i * LANES, LANES)] = inc - h + carry
    hist[pl.ds(i * LANES, LANES)] = _zero16()
    return carry + jnp.sum(h)

  def hist_update(d):
    c, last = plsc.scan_count(d)
    plsc.addupdate_scatter(hist, [d], c, mask=last)

  def digit(u, shift, mask):
    d = u if shift == 0 else lax.shift_right_logical(u, shift)
    return lax.bitwise_and(d, jnp.int32(mask))

  def map_and_hist0(i, c):
    sl = pl.ds(i * LANES, LANES)
    u = _to_monotone(k0[sl])
    k0[sl] = u
    hist_update(digit(u, 0, 0x7FF))
    return c

  def make_perm(src, dst, shift, mask, next_shift, next_mask, finalize):
    def perm(i, c):
      u = src[pl.ds(i * LANES, LANES)]
      d = digit(u, shift, mask)
      cnt, last = plsc.scan_count(d)
      base = plsc.load_gather(offs, [d])
      pos = base + cnt - 1
      v = _from_monotone(u) if finalize else u
      plsc.store_scatter(dst, [pos], v)
      plsc.addupdate_scatter(offs, [d], cnt, mask=last)
      if next_shift is not None:
        hist_update(digit(u, next_shift, next_mask))
      return c
    return perm

  for r in range(ROWS_PER_WORKER):
    row = wid * ROWS_PER_WORKER + r
    pltpu.sync_copy(in_hbm.at[row], k0)
    lax.fori_loop(0, 2048 // LANES, zero_hist, jnp.int32(0))
    lax.fori_loop(0, NV, map_and_hist0, jnp.int32(0))
    lax.fori_loop(0, 2048 // LANES, scan_hist, jnp.int32(0))
    lax.fori_loop(0, NV, make_perm(k0, k1, 0, 0x7FF, 11, 0x7FF, False),
                  jnp.int32(0))
    lax.fori_loop(0, 2048 // LANES, scan_hist, jnp.int32(0))
    lax.fori_loop(0, NV, make_perm(k1, k0, 11, 0x7FF, 22, 0x3FF, False),
                  jnp.int32(0))
    lax.fori_loop(0, 2048 // LANES, scan_hist, jnp.int32(0))
    lax.fori_loop(0, NV, make_perm(k0, k1, 22, 0x3FF, None, None, True),
                  jnp.int32(0))
    pltpu.sync_copy(k1, out_hbm.at[row])


@jax.jit
def kernel(inputs):
  xi = lax.bitcast_convert_type(inputs, jnp.int32)
  mesh = plsc.VectorSubcoreMesh(
      core_axis_name="c", subcore_axis_name="s", num_cores=NC,
      num_subcores=NS)
  sorted_i = pl.kernel(
      _sort_body,
      out_type=jax.ShapeDtypeStruct((ROWS, N), jnp.int32),
      mesh=mesh,
      scratch_types=[
          pltpu.VMEM((N,), jnp.int32),
          pltpu.VMEM((N,), jnp.int32),
          pltpu.VMEM((2048,), jnp.int32),
          pltpu.VMEM((2048,), jnp.int32),
      ],
      compiler_params=pltpu.CompilerParams(needs_layout_passes=False),
  )(xi)
  return lax.bitcast_convert_type(sorted_i, jnp.float32)
